# Initial kernel scaffold; baseline (speedup 1.0000x reference)
import jax
import jax.numpy as jnp
from jax.experimental import pallas as pl


def kernel(line_dist_mat, angle_mat, boundaries, r_pos_emb_table):
    raise NotImplementedError("write your pallas kernel here")


# speedup vs baseline: 27.3064x; 27.3064x over previous
import functools

import jax
import jax.numpy as jnp
from jax import lax
from jax.experimental import pallas as pl
from jax.experimental.pallas import tpu as pltpu
from jax.experimental.pallas import tpu_sc as plsc

EMB = 16
NBINS = 101

NW = 32
CHUNK = 2048
IDX_ROWS = CHUNK // 128


def _idx_body(p_ref, d_ref, a_ref, o_ref):
    b0 = p_ref[0]
    inv_step = p_ref[1]
    step = p_ref[2]
    d = d_ref[...]
    a = a_ref[...]
    u = d * jnp.cos(a)
    v = d * jnp.sin(a)

    def bucket(x):
        y = (x - b0) * inv_step
        y = jnp.clip(y, -1.0, float(NBINS + 2))
        g = y.astype(jnp.int32)
        gf = g.astype(jnp.float32)
        hi = b0 + gf * step
        lo = hi - step
        idx = g + jnp.where((g <= NBINS - 1) & (hi < x), 1, 0)
        idx = idx - jnp.where((g >= 1) & (lo >= x), 1, 0)
        return jnp.clip(idx, 0, NBINS)

    o_ref[...] = bucket(v) * NBINS + bucket(u)


def _compute_indices(params, dist2d, angle2d, rows, cols, grid_n):
    blk = rows // grid_n
    return pl.pallas_call(
        _idx_body,
        grid=(grid_n,),
        in_specs=[
            pl.BlockSpec(memory_space=pltpu.SMEM),
            pl.BlockSpec((blk, cols), lambda i: (i, 0)),
            pl.BlockSpec((blk, cols), lambda i: (i, 0)),
        ],
        out_specs=pl.BlockSpec((blk, cols), lambda i: (i, 0)),
        out_shape=jax.ShapeDtypeStruct((rows, cols), jnp.int32),
    )(params, dist2d, angle2d)


def _sc_gather(idx2d, table, n):
    per_w = n // NW
    chunks = per_w // CHUNK
    row_stride = per_w // 128
    mesh = plsc.VectorSubcoreMesh(core_axis_name="c", subcore_axis_name="s")

    @functools.partial(
        pl.kernel,
        out_type=jax.ShapeDtypeStruct((n, EMB), jnp.float32),
        mesh=mesh,
        scratch_types=[
            pltpu.VMEM((IDX_ROWS, 128), jnp.int32),
            pltpu.VMEM((CHUNK, EMB), jnp.float32),
            pltpu.SemaphoreType.DMA,
        ],
        compiler_params=pltpu.CompilerParams(use_tc_tiling_on_sc=False),
    )
    def body(idx_hbm, table_hbm, out_hbm, idx_v, rows_v, sem):
        wid = lax.axis_index("s") * 2 + lax.axis_index("c")
        w_row = wid * row_stride
        w_elt = wid * per_w

        def chunk_body(c, carry):
            row0 = w_row + c * IDX_ROWS
            e0 = w_elt + c * CHUNK
            pltpu.sync_copy(idx_hbm.at[pl.ds(row0, IDX_ROWS)], idx_v)
            copies = [
                pltpu.async_copy(
                    table_hbm.at[idx_v.at[j]],
                    rows_v.at[pl.ds(j * 128, 128)],
                    sem,
                )
                for j in range(IDX_ROWS)
            ]
            for cp in copies:
                cp.wait()
            pltpu.sync_copy(rows_v, out_hbm.at[pl.ds(e0, CHUNK)])
            return carry

        lax.fori_loop(0, chunks, chunk_body, 0)

    return body(idx2d, table)


def kernel(line_dist_mat, angle_mat, boundaries, r_pos_emb_table):
    b, _, l = line_dist_mat.shape
    n = b * l
    cols = 1024
    rows = n // cols

    dist2d = line_dist_mat[:, 1, :].reshape(rows, cols)
    angle2d = angle_mat[:, 1, :].reshape(rows, cols)
    b0 = boundaries[0]
    step = boundaries[1] - boundaries[0]
    params = jnp.stack([b0, 1.0 / step, step])

    idx = _compute_indices(params, dist2d, angle2d, rows, cols, grid_n=16)
    idx2d = idx.reshape(n // 128, 128)
    out = _sc_gather(idx2d, r_pos_emb_table, n)
    return out.reshape(b, l, EMB)


# speedup vs baseline: 178.9896x; 6.5549x over previous
import functools

import jax
import jax.numpy as jnp
from jax import lax
from jax.experimental import pallas as pl
from jax.experimental.pallas import tpu as pltpu
from jax.experimental.pallas import tpu_sc as plsc

EMB = 16
HALF = 8
NBINS = 101

NSUB = 16
CHUNK = 2048
IDX_ROWS = CHUNK // 128


def _idx_body(p_ref, d_ref, a_ref, o_ref):
    b0 = p_ref[0]
    inv_step = p_ref[1]
    step = p_ref[2]
    d = d_ref[...]
    a = a_ref[...]
    u = d * jnp.cos(a)
    v = d * jnp.sin(a)

    def bucket(x):
        y = (x - b0) * inv_step
        y = jnp.clip(y, -1.0, float(NBINS + 2))
        g = y.astype(jnp.int32)
        gf = g.astype(jnp.float32)
        hi = b0 + gf * step
        lo = hi - step
        idx = g + jnp.where((g <= NBINS - 1) & (hi < x), 1, 0)
        idx = idx - jnp.where((g >= 1) & (lo >= x), 1, 0)
        return jnp.clip(idx, 0, NBINS)

    o_ref[...] = bucket(v) * NBINS + bucket(u)


def _compute_indices(params, dist2d, angle2d, rows, grid_n):
    blk = rows // grid_n
    return pl.pallas_call(
        _idx_body,
        grid=(grid_n,),
        in_specs=[
            pl.BlockSpec(memory_space=pltpu.SMEM),
            pl.BlockSpec((blk, 128), lambda i: (i, 0)),
            pl.BlockSpec((blk, 128), lambda i: (i, 0)),
        ],
        out_specs=pl.BlockSpec((blk, 128), lambda i: (i, 0)),
        out_shape=jax.ShapeDtypeStruct((rows, 128), jnp.int32),
    )(params, dist2d, angle2d)


def _sc_gather(idx2d, tbl_halves, n, vrows):
    per_s = n // NSUB
    chunks = per_s // CHUNK
    row_stride = per_s // 128
    mesh = plsc.VectorSubcoreMesh(core_axis_name="c", subcore_axis_name="s")

    @functools.partial(
        pl.kernel,
        out_type=jax.ShapeDtypeStruct((n, EMB), jnp.float32),
        mesh=mesh,
        scratch_types=[
            pltpu.VMEM((vrows, HALF), jnp.float32),
            pltpu.VMEM((IDX_ROWS, 128), jnp.int32),
            pltpu.VMEM((CHUNK, HALF), jnp.float32),
        ],
        compiler_params=pltpu.CompilerParams(
            use_tc_tiling_on_sc=False, needs_layout_passes=False
        ),
    )
    def body(idx_hbm, tbl_hbm, out_hbm, tbl_v, idx_v, out_v):
        c = lax.axis_index("c")
        s = lax.axis_index("s")
        pltpu.sync_copy(tbl_hbm.at[c], tbl_v)
        w_row = s * row_stride
        w_elt = s * per_s
        col0 = c * HALF
        iota = lax.iota(jnp.int32, 16)
        cols = [jnp.full((16,), cc, jnp.int32) for cc in range(HALF)]

        def chunk_body(k, carry):
            row0 = w_row + k * IDX_ROWS
            e0 = w_elt + k * CHUNK
            pltpu.sync_copy(idx_hbm.at[pl.ds(row0, IDX_ROWS)], idx_v)

            def grp(r, carry2):
                base_r = r * 128 + iota
                for j in range(8):
                    iv = idx_v[r, pl.ds(j * 16, 16)]
                    rows_loc = base_r + j * 16
                    for cc in range(HALF):
                        vals = plsc.load_gather(tbl_v, [iv, cols[cc]])
                        plsc.store_scatter(out_v, [rows_loc, cols[cc]], vals)
                return carry2

            lax.fori_loop(0, IDX_ROWS, grp, 0)
            pltpu.sync_copy(
                out_v, out_hbm.at[pl.ds(e0, CHUNK), pl.ds(col0, HALF)]
            )
            return carry

        lax.fori_loop(0, chunks, chunk_body, 0)

    return body(idx2d, tbl_halves)


def kernel(line_dist_mat, angle_mat, boundaries, r_pos_emb_table):
    b, _, l = line_dist_mat.shape
    n = b * l
    rows = n // 128

    dist2d = line_dist_mat[:, 1, :].reshape(rows, 128)
    angle2d = angle_mat[:, 1, :].reshape(rows, 128)
    b0 = boundaries[0]
    step = boundaries[1] - boundaries[0]
    params = jnp.stack([b0, 1.0 / step, step])

    vrows = r_pos_emb_table.shape[0]
    tbl_halves = jnp.stack(
        [r_pos_emb_table[:, :HALF], r_pos_emb_table[:, HALF:]]
    )

    idx = _compute_indices(params, dist2d, angle2d, rows, grid_n=16)
    out = _sc_gather(idx, tbl_halves, n, vrows)
    return out.reshape(b, l, EMB)


# speedup vs baseline: 192.6408x; 1.0763x over previous
import functools

import jax
import jax.numpy as jnp
from jax import lax
from jax.experimental import pallas as pl
from jax.experimental.pallas import tpu as pltpu
from jax.experimental.pallas import tpu_sc as plsc

EMB = 16
HALF = 8
NBINS = 101

NSUB = 16
CPAD = 256
RPC = 8


_TWO_OVER_PI = 0.6366197723675814
_PIO2_HI = 1.5707963705062866
_PIO2_LO = -4.371139000186241e-8
_MAGIC = 12582912.0
_SIN1, _SIN2, _SIN3 = -1.6666654611e-1, 8.3321608736e-3, -1.9515295891e-4
_COS1, _COS2, _COS3 = 4.166664568298827e-2, -1.388731625493765e-3, \
    2.443315711809948e-5


def _sincos(a):
    t = jnp.clip(a * _TWO_OVER_PI, -6.0e5, 6.0e5)
    kf = (t + _MAGIC) - _MAGIC
    r = a - kf * _PIO2_HI
    r = r - kf * _PIO2_LO
    r = jnp.clip(r, -1.0, 1.0)
    r2 = r * r
    sin_r = r + r * r2 * (_SIN1 + r2 * (_SIN2 + r2 * _SIN3))
    cos_r = 1.0 - 0.5 * r2 + r2 * r2 * (_COS1 + r2 * (_COS2 + r2 * _COS3))
    q = kf.astype(jnp.int32) & 3
    swap = (q & 1) == 1
    s1 = jnp.where(swap, cos_r, sin_r)
    c1 = jnp.where(swap, sin_r, cos_r)
    sin_a = jnp.where(q >= 2, -s1, s1)
    cos_a = jnp.where((q == 1) | (q == 2), -c1, c1)
    return sin_a, cos_a


def _idx_body(p_ref, d_ref, a_ref, o_ref, d_s, a_s):
    b0 = p_ref[0]
    inv_step = p_ref[1]
    step = p_ref[2]
    d_s[...] = d_ref[:, 1, :]
    a_s[...] = a_ref[:, 1, :]
    d = d_s[...]
    a = a_s[...]
    sin_a, cos_a = _sincos(a)
    u = d * cos_a
    v = d * sin_a

    def bucket(x):
        y = (x - b0) * inv_step
        y = jnp.clip(y, -1.0, float(NBINS + 2))
        g = y.astype(jnp.int32)
        gf = g.astype(jnp.float32)
        hi = b0 + gf * step
        lo = hi - step
        idx = g + jnp.where((g <= NBINS - 1) & (hi < x), 1, 0)
        idx = idx - jnp.where((g >= 1) & (lo >= x), 1, 0)
        return jnp.clip(idx, 0, NBINS)

    l = d.shape[1]
    o_ref[:, :l] = bucket(v) * NBINS + bucket(u)


def _compute_indices(params, line_dist_mat, angle_mat, grid_n):
    b, two, l = line_dist_mat.shape
    blk = b // grid_n
    return pl.pallas_call(
        _idx_body,
        grid=(grid_n,),
        in_specs=[
            pl.BlockSpec(memory_space=pltpu.SMEM),
            pl.BlockSpec((blk, two, l), lambda i: (i, 0, 0)),
            pl.BlockSpec((blk, two, l), lambda i: (i, 0, 0)),
        ],
        out_specs=pl.BlockSpec((blk, CPAD), lambda i: (i, 0)),
        out_shape=jax.ShapeDtypeStruct((b, CPAD), jnp.int32),
        scratch_shapes=[
            pltpu.VMEM((blk, l), jnp.float32),
            pltpu.VMEM((blk, l), jnp.float32),
        ],
    )(params, line_dist_mat, angle_mat)


def _sc_gather(idx2d, tbl_halves, b, l, vrows):
    rows_per_s = b // NSUB
    chunks = rows_per_s // RPC
    offs = list(range(0, l - 16, 16)) + [l - 16]
    mesh = plsc.VectorSubcoreMesh(core_axis_name="c", subcore_axis_name="s")

    @functools.partial(
        pl.kernel,
        out_type=jax.ShapeDtypeStruct((b, l, EMB), jnp.float32),
        mesh=mesh,
        scratch_types=[
            pltpu.VMEM((vrows * HALF,), jnp.float32),
            pltpu.VMEM((RPC, CPAD), jnp.int32),
            pltpu.VMEM((RPC, CPAD), jnp.int32),
            pltpu.VMEM((RPC, l, HALF), jnp.float32),
            pltpu.VMEM((RPC, l, HALF), jnp.float32),
            pltpu.SemaphoreType.DMA,
            pltpu.SemaphoreType.DMA,
            pltpu.SemaphoreType.DMA,
            pltpu.SemaphoreType.DMA,
        ],
        compiler_params=pltpu.CompilerParams(
            use_tc_tiling_on_sc=False, needs_layout_passes=False
        ),
    )
    def body(idx_hbm, tbl_hbm, out_hbm, tbl_v, idx_v0, idx_v1, out_v0,
             out_v1, sem_i0, sem_i1, sem_o0, sem_o1):
        c = lax.axis_index("c")
        s = lax.axis_index("s")
        pltpu.sync_copy(tbl_hbm.at[c], tbl_v)
        row_base = s * rows_per_s
        col0 = c * HALF
        iota = lax.iota(jnp.int32, 16)
        last = chunks - 1

        def idx_copy(k, idx_v, sem):
            r0 = row_base + k * RPC
            pltpu.async_copy(idx_hbm.at[pl.ds(r0, RPC)], idx_v, sem)

        def idx_wait(idx_v, sem):
            pltpu.make_async_copy(idx_hbm.at[pl.ds(row_base, RPC)], idx_v,
                                  sem).wait()

        def out_dma(k, out_v, sem):
            r0 = row_base + k * RPC
            pltpu.async_copy(
                out_v, out_hbm.at[pl.ds(r0, RPC), :, pl.ds(col0, HALF)], sem
            )

        def out_wait(out_v, sem):
            pltpu.make_async_copy(
                out_v, out_hbm.at[pl.ds(row_base, RPC), :,
                                  pl.ds(col0, HALF)], sem
            ).wait()

        def gather(idx_v, out_v):
            def row(r, carry):
                rv = jnp.full((16,), r, jnp.int32)
                for off in offs:
                    iv = idx_v[r, pl.ds(off, 16)]
                    im8 = iv * HALF
                    lv = off + iota
                    for cc in range(HALF):
                        vals = plsc.load_gather(tbl_v, [im8 + cc])
                        plsc.store_scatter(
                            out_v, [rv, lv, jnp.full((16,), cc, jnp.int32)],
                            vals,
                        )
                return carry

            lax.fori_loop(0, RPC, row, 0)

        idx_copy(0, idx_v0, sem_i0)

        def pair(kk, carry):
            k0 = kk * 2
            idx_wait(idx_v0, sem_i0)
            idx_copy(k0 + 1, idx_v1, sem_i1)

            @pl.when(kk > 0)
            def _():
                out_wait(out_v0, sem_o0)

            gather(idx_v0, out_v0)
            out_dma(k0, out_v0, sem_o0)
            idx_wait(idx_v1, sem_i1)
            idx_copy(jnp.minimum(k0 + 2, last), idx_v0, sem_i0)

            @pl.when(kk > 0)
            def _():
                out_wait(out_v1, sem_o1)

            gather(idx_v1, out_v1)
            out_dma(k0 + 1, out_v1, sem_o1)
            return carry

        lax.fori_loop(0, chunks // 2, pair, 0)
        idx_wait(idx_v0, sem_i0)
        out_wait(out_v0, sem_o0)
        out_wait(out_v1, sem_o1)

    return body(idx2d, tbl_halves)


def kernel(line_dist_mat, angle_mat, boundaries, r_pos_emb_table):
    b, _, l = line_dist_mat.shape

    b0 = boundaries[0]
    step = boundaries[1] - boundaries[0]
    params = jnp.stack([b0, 1.0 / step, step])

    vrows = r_pos_emb_table.shape[0]
    tbl_halves = jnp.stack(
        [
            r_pos_emb_table[:, :HALF].reshape(-1),
            r_pos_emb_table[:, HALF:].reshape(-1),
        ]
    )

    idx = _compute_indices(params, line_dist_mat, angle_mat, grid_n=16)
    return _sc_gather(idx, tbl_halves, b, l, vrows)


# speedup vs baseline: 980.0065x; 5.0872x over previous
import functools

import jax
import jax.numpy as jnp
from jax import lax
from jax.experimental import pallas as pl
from jax.experimental.pallas import tpu as pltpu
from jax.experimental.pallas import tpu_sc as plsc

EMB = 16
HALF = 8
NBINS = 101
NSUB = 16

_TWO_OVER_PI = 0.6366197723675814
_PIO2_HI = 1.5707963705062866
_PIO2_LO = -4.371139000186241e-8
_MAGIC = 12582912.0
_SIN1, _SIN2, _SIN3 = -1.6666654611e-1, 8.3321608736e-3, -1.9515295891e-4
_COS1, _COS2, _COS3 = 4.166664568298827e-2, -1.388731625493765e-3, \
    2.443315711809948e-5


def _sincos(a):
    t = jnp.clip(a * _TWO_OVER_PI, -6.0e5, 6.0e5)
    kf = (t + _MAGIC) - _MAGIC
    r = a - kf * _PIO2_HI
    r = r - kf * _PIO2_LO
    r = jnp.clip(r, -1.0, 1.0)
    r2 = r * r
    sin_r = r + r * r2 * (_SIN1 + r2 * (_SIN2 + r2 * _SIN3))
    cos_r = 1.0 - 0.5 * r2 + r2 * r2 * (_COS1 + r2 * (_COS2 + r2 * _COS3))
    q = kf.astype(jnp.int32) & 3
    swap = (q & 1) == 1
    s1 = jnp.where(swap, cos_r, sin_r)
    c1 = jnp.where(swap, sin_r, cos_r)
    sin_a = jnp.where(q >= 2, -s1, s1)
    cos_a = jnp.where((q == 1) | (q == 2), -c1, c1)
    return sin_a, cos_a


def _idx_body(p_ref, d_ref, a_ref, o_ref):
    b0 = p_ref[0]
    inv_step = p_ref[1]
    step = p_ref[2]
    d = d_ref[0]
    a = a_ref[0]
    sin_a, cos_a = _sincos(a)
    u = d * cos_a
    v = d * sin_a

    def bucket(x):
        y = (x - b0) * inv_step
        y = jnp.clip(y, -1.0, float(NBINS + 2))
        g = y.astype(jnp.int32)
        gf = g.astype(jnp.float32)
        hi = b0 + gf * step
        lo = hi - step
        idx = g + jnp.where((g <= NBINS - 1) & (hi < x), 1, 0)
        idx = idx - jnp.where((g >= 1) & (lo >= x), 1, 0)
        return jnp.clip(idx, 0, NBINS)

    idx = bucket(v) * NBINS + bucket(u)
    nl = idx.shape[0] // 8
    nc = idx.shape[1] // 128
    for ll in range(nl):
        for cb in range(nc):
            o_ref[ll, cb] = idx[8 * ll:8 * ll + 8, 128 * cb:128 * cb + 128]


def _compute_indices(params, distT, angleT, grid_n):
    two, l, b = distT.shape
    bw = b // grid_n
    return pl.pallas_call(
        _idx_body,
        grid=(grid_n,),
        in_specs=[
            pl.BlockSpec(memory_space=pltpu.SMEM),
            pl.BlockSpec((1, l, bw), lambda i: (1, 0, i)),
            pl.BlockSpec((1, l, bw), lambda i: (1, 0, i)),
        ],
        out_specs=pl.BlockSpec(
            (l // 8, bw // 128, 8, 128), lambda i: (0, i, 0, 0)
        ),
        out_shape=jax.ShapeDtypeStruct(
            (l // 8, b // 128, 8, 128), jnp.int32
        ),
    )(params, distT, angleT)


def _sc_gather(idx4, tbl8, b, l, vrows):
    nl8 = l // 8
    mesh = plsc.VectorSubcoreMesh(core_axis_name="c", subcore_axis_name="s")

    @functools.partial(
        pl.kernel,
        out_type=jax.ShapeDtypeStruct((l, 2, b // 128, HALF, 128),
                                      jnp.float32),
        mesh=mesh,
        scratch_types=(
            [pltpu.VMEM((vrows,), jnp.float32) for _ in range(HALF)]
            + [pltpu.VMEM((HALF, 8, 128), jnp.int32) for _ in range(2)]
            + [pltpu.VMEM((HALF, HALF, 128), jnp.float32) for _ in range(2)]
            + [pltpu.SemaphoreType.DMA for _ in range(4)]
        ),
        compiler_params=pltpu.CompilerParams(
            use_tc_tiling_on_sc=False, needs_layout_passes=False
        ),
    )
    def body(idx_hbm, tbl_hbm, out_hbm, t0, t1, t2, t3, t4, t5, t6, t7,
             ix0, ix1, ov0, ov1, si0, si1, so0, so1):
        c = lax.axis_index("c")
        s = lax.axis_index("s")
        tv = [t0, t1, t2, t3, t4, t5, t6, t7]
        ixs = [ix0, ix1]
        sis = [si0, si1]
        for c8 in range(HALF):
            pltpu.sync_copy(tbl_hbm.at[c, c8], tv[c8])

        def idx_start(ll, ix, sem):
            pltpu.async_copy(idx_hbm.at[ll, pl.ds(s * 8, 8)], ix, sem)

        def idx_wait(ix, sem):
            pltpu.make_async_copy(
                idx_hbm.at[0, pl.ds(s * 8, 8)], ix, sem
            ).wait()

        def out_start(lv, ov, sem):
            pltpu.async_copy(
                ov, out_hbm.at[lv, c, pl.ds(s * 8, 8)], sem
            )

        def out_wait(ov, sem):
            pltpu.make_async_copy(
                ov, out_hbm.at[0, c, pl.ds(s * 8, 8)], sem
            ).wait()

        def gather_l(ix, li, ov):
            def ci_body(ci, carry):
                for p in range(8):
                    iv = ix[ci, li, pl.ds(16 * p, 16)]
                    for c8 in range(HALF):
                        ov[ci, c8, pl.ds(16 * p, 16)] = \
                            plsc.load_gather(tv[c8], [iv])
                return carry

            lax.fori_loop(0, HALF, ci_body, 0)

        def do_lrow(ll, ixb):
            idx_wait(ixs[ixb], sis[ixb])

            def li_pair(j, carry):
                li0 = j * 2
                out_wait(ov0, so0)
                gather_l(ixs[ixb], li0, ov0)
                out_start(ll * 8 + li0, ov0, so0)
                out_wait(ov1, so1)
                gather_l(ixs[ixb], li0 + 1, ov1)
                out_start(ll * 8 + li0 + 1, ov1, so1)
                return carry

            lax.fori_loop(0, 4, li_pair, 0)

        idx_start(0, ix0, si0)
        idx_start(1, ix1, si1)
        out_start(0, ov0, so0)
        out_start(1, ov1, so1)

        def pair(kk, carry):
            ll0 = kk * 2
            do_lrow(ll0, 0)
            idx_start(ll0 + 2, ix0, si0)
            do_lrow(ll0 + 1, 1)
            idx_start(jnp.minimum(ll0 + 3, nl8 - 1), ix1, si1)
            return carry

        lax.fori_loop(0, (nl8 - 1) // 2, pair, 0)
        do_lrow(nl8 - 1, 0)
        idx_wait(ix1, si1)
        out_wait(ov0, so0)
        out_wait(ov1, so1)

    return body(idx4, tbl8)


def kernel(line_dist_mat, angle_mat, boundaries, r_pos_emb_table):
    b, _, l = line_dist_mat.shape

    b0 = boundaries[0]
    step = boundaries[1] - boundaries[0]
    params = jnp.stack([b0, 1.0 / step, step])

    vrows = r_pos_emb_table.shape[0]
    tbl8 = r_pos_emb_table.T.reshape(2, HALF, vrows)

    distT = jnp.transpose(line_dist_mat, (1, 2, 0))
    angleT = jnp.transpose(angle_mat, (1, 2, 0))
    idx4 = _compute_indices(params, distT, angleT, grid_n=16)
    out5 = _sc_gather(idx4, tbl8, b, l, vrows)
    return out5.transpose(2, 4, 0, 1, 3).reshape(b, l, EMB)


# speedup vs baseline: 2909.2948x; 2.9686x over previous
import functools

import jax
import jax.numpy as jnp
from jax import lax
from jax.experimental import pallas as pl
from jax.experimental.pallas import tpu as pltpu
from jax.experimental.pallas import tpu_sc as plsc

EMB = 16
HALF = 8
NBINS = 101
NSUB = 16

_TWO_OVER_PI = 0.6366197723675814
_PIO2_HI = 1.5707963705062866
_PIO2_LO = -4.371139000186241e-8
_MAGIC = 12582912.0
_SIN1, _SIN2, _SIN3 = -1.6666654611e-1, 8.3321608736e-3, -1.9515295891e-4
_COS1, _COS2, _COS3 = 4.166664568298827e-2, -1.388731625493765e-3, \
    2.443315711809948e-5


def _sincos(a):
    t = jnp.clip(a * _TWO_OVER_PI, -6.0e5, 6.0e5)
    kf = (t + _MAGIC) - _MAGIC
    r = a - kf * _PIO2_HI
    r = r - kf * _PIO2_LO
    r = jnp.clip(r, -1.0, 1.0)
    r2 = r * r
    sin_r = r + r * r2 * (_SIN1 + r2 * (_SIN2 + r2 * _SIN3))
    cos_r = 1.0 - 0.5 * r2 + r2 * r2 * (_COS1 + r2 * (_COS2 + r2 * _COS3))
    q = kf.astype(jnp.int32) & 3
    swap = (q & 1) == 1
    s1 = jnp.where(swap, cos_r, sin_r)
    c1 = jnp.where(swap, sin_r, cos_r)
    sin_a = jnp.where(q >= 2, -s1, s1)
    cos_a = jnp.where((q == 1) | (q == 2), -c1, c1)
    return sin_a, cos_a


def _idx_body(p_ref, d_ref, a_ref, o_ref):
    b0 = p_ref[0]
    inv_step = p_ref[1]
    step = p_ref[2]
    d = d_ref[0]
    a = a_ref[0]
    sin_a, cos_a = _sincos(a)
    u = d * cos_a
    v = d * sin_a

    def bucket(x):
        y = (x - b0) * inv_step
        y = jnp.clip(y, -1.0, float(NBINS + 2))
        g = y.astype(jnp.int32)
        gf = g.astype(jnp.float32)
        hi = b0 + gf * step
        lo = hi - step
        idx = g + jnp.where((g <= NBINS - 1) & (hi < x), 1, 0)
        idx = idx - jnp.where((g >= 1) & (lo >= x), 1, 0)
        return jnp.clip(idx, 0, NBINS)

    idx = bucket(v) * NBINS + bucket(u)
    nl = idx.shape[0] // 8
    nc = idx.shape[1] // 128
    for ll in range(nl):
        for cb in range(nc):
            o_ref[ll, cb] = idx[8 * ll:8 * ll + 8, 128 * cb:128 * cb + 128]


def _compute_indices(params, distT, angleT, grid_n):
    two, l, b = distT.shape
    bw = b // grid_n
    return pl.pallas_call(
        _idx_body,
        grid=(grid_n,),
        in_specs=[
            pl.BlockSpec(memory_space=pltpu.SMEM),
            pl.BlockSpec((1, l, bw), lambda i: (1, 0, i)),
            pl.BlockSpec((1, l, bw), lambda i: (1, 0, i)),
        ],
        out_specs=pl.BlockSpec(
            (l // 8, bw // 128, 8, 128), lambda i: (0, i, 0, 0)
        ),
        out_shape=jax.ShapeDtypeStruct(
            (l // 8, b // 128, 8, 128), jnp.int32
        ),
    )(params, distT, angleT)


def _sc_gather(idx4, tbl8, b, l, vrows):
    nl8 = l // 8
    mesh = plsc.VectorSubcoreMesh(core_axis_name="c", subcore_axis_name="s")

    @functools.partial(
        pl.kernel,
        out_type=jax.ShapeDtypeStruct((l, 2, b // 128, HALF, 128),
                                      jnp.float32),
        mesh=mesh,
        scratch_types=(
            [pltpu.VMEM((vrows,), jnp.float32) for _ in range(HALF)]
            + [pltpu.VMEM((HALF, 8, 128), jnp.int32) for _ in range(2)]
            + [pltpu.VMEM((HALF, HALF, 128), jnp.float32) for _ in range(2)]
            + [pltpu.SemaphoreType.DMA for _ in range(4)]
        ),
        compiler_params=pltpu.CompilerParams(
            use_tc_tiling_on_sc=False, needs_layout_passes=False
        ),
    )
    def body(idx_hbm, tbl_hbm, out_hbm, t0, t1, t2, t3, t4, t5, t6, t7,
             ix0, ix1, ov0, ov1, si0, si1, so0, so1):
        c = lax.axis_index("c")
        s = lax.axis_index("s")
        tv = [t0, t1, t2, t3, t4, t5, t6, t7]
        ixs = [ix0, ix1]
        sis = [si0, si1]
        for c8 in range(HALF):
            pltpu.sync_copy(tbl_hbm.at[c, c8], tv[c8])

        def idx_start(ll, ix, sem):
            pltpu.async_copy(idx_hbm.at[ll, pl.ds(s * 8, 8)], ix, sem)

        def idx_wait(ix, sem):
            pltpu.make_async_copy(
                idx_hbm.at[0, pl.ds(s * 8, 8)], ix, sem
            ).wait()

        def out_start(lv, ov, sem):
            pltpu.async_copy(
                ov, out_hbm.at[lv, c, pl.ds(s * 8, 8)], sem
            )

        def out_wait(ov, sem):
            pltpu.make_async_copy(
                ov, out_hbm.at[0, c, pl.ds(s * 8, 8)], sem
            ).wait()

        def gather_l(ix, li, ov):
            iv0 = ix[0, li, pl.ds(0, 16)]

            def mm_body(ci, mm):
                mn, mx = mm
                for p in range(8):
                    iv = ix[ci, li, pl.ds(16 * p, 16)]
                    mn = jnp.minimum(mn, iv)
                    mx = jnp.maximum(mx, iv)
                return (mn, mx)

            mn, mx = lax.fori_loop(0, HALF, mm_body, (iv0, iv0))
            uniform = jnp.min(mn) == jnp.max(mx)

            def uni_path(carry):
                vals = [plsc.load_gather(tv[c8], [iv0]) for c8 in range(HALF)]

                def st_body(ci, carry2):
                    for p in range(8):
                        for c8 in range(HALF):
                            ov[ci, c8, pl.ds(16 * p, 16)] = vals[c8]
                    return carry2

                return lax.fori_loop(0, HALF, st_body, 0)

            def gen_path(carry):
                def ci_body(ci, carry2):
                    for p in range(8):
                        iv = ix[ci, li, pl.ds(16 * p, 16)]
                        for c8 in range(HALF):
                            ov[ci, c8, pl.ds(16 * p, 16)] = \
                                plsc.load_gather(tv[c8], [iv])
                    return carry2

                return lax.fori_loop(0, HALF, ci_body, 0)

            lax.cond(uniform, uni_path, gen_path, 0)

        def do_lrow(ll, ixb):
            idx_wait(ixs[ixb], sis[ixb])

            def li_pair(j, carry):
                li0 = j * 2
                out_wait(ov0, so0)
                gather_l(ixs[ixb], li0, ov0)
                out_start(ll * 8 + li0, ov0, so0)
                out_wait(ov1, so1)
                gather_l(ixs[ixb], li0 + 1, ov1)
                out_start(ll * 8 + li0 + 1, ov1, so1)
                return carry

            lax.fori_loop(0, 4, li_pair, 0)

        idx_start(0, ix0, si0)
        idx_start(1, ix1, si1)
        out_start(0, ov0, so0)
        out_start(1, ov1, so1)

        def pair(kk, carry):
            ll0 = kk * 2
            do_lrow(ll0, 0)
            idx_start(ll0 + 2, ix0, si0)
            do_lrow(ll0 + 1, 1)
            idx_start(jnp.minimum(ll0 + 3, nl8 - 1), ix1, si1)
            return carry

        lax.fori_loop(0, (nl8 - 1) // 2, pair, 0)
        do_lrow(nl8 - 1, 0)
        idx_wait(ix1, si1)
        out_wait(ov0, so0)
        out_wait(ov1, so1)

    return body(idx4, tbl8)


def kernel(line_dist_mat, angle_mat, boundaries, r_pos_emb_table):
    b, _, l = line_dist_mat.shape

    b0 = boundaries[0]
    step = boundaries[1] - boundaries[0]
    params = jnp.stack([b0, 1.0 / step, step])

    vrows = r_pos_emb_table.shape[0]
    tbl8 = r_pos_emb_table.T.reshape(2, HALF, vrows)

    distT = jnp.transpose(line_dist_mat, (1, 2, 0))
    angleT = jnp.transpose(angle_mat, (1, 2, 0))
    idx4 = _compute_indices(params, distT, angleT, grid_n=16)
    out5 = _sc_gather(idx4, tbl8, b, l, vrows)
    return out5.transpose(2, 4, 0, 1, 3).reshape(b, l, EMB)
